# baseline (speedup 1.0000x reference)
import jax
import jax.numpy as jnp
from jax import lax
from jax.experimental import pallas as pl
from jax.experimental.pallas import tpu as pltpu
from jax.experimental.pallas import tpu_sc as plsc

_VOCAB = 100000
_HALF = _VOCAB // 2
_DEPTH = 4
_NCHILD = 64
_B, _T = 4, 32
_ROWS = _B * _T
_UPD = _DEPTH * _NCHILD
_NW = 32
_ROWS_PER_W = _ROWS // _NW
_FILL_UNROLL = 25
_FILL_STEPS = _HALF // (16 * _FILL_UNROLL)


def _sc_body(tok_hbm, cnt_hbm, w_hbm, b_hbm, out_hbm,
             buf0, buf1, tok_v0, tok_v1, cnt_v0, cnt_v1, w_v, b_v, sem0, sem1):
    tok_vs = [tok_v0, tok_v1]
    cnt_vs = [cnt_v0, cnt_v1]
    wid = lax.axis_index("s") * 2 + lax.axis_index("c")
    pltpu.sync_copy(w_hbm, w_v)
    pltpu.sync_copy(b_hbm, b_v)
    bv = b_v[...]
    lane = lax.iota(jnp.int32, 16)
    bufs = [buf0, buf1]
    sems = [sem0, sem1]

    def _fill(buf):
        def body(i, c):
            base = i * (16 * _FILL_UNROLL)
            for u in range(_FILL_UNROLL):
                buf[pl.ds(base + u * 16, 16)] = bv
            return c
        lax.fori_loop(0, _FILL_STEPS, body, 0)

    _fill(buf0)
    _fill(buf1)

    handles = [None, None]
    for r in range(_ROWS_PER_W):
        row = wid * _ROWS_PER_W + r
        p = r % 2
        tok_v = tok_vs[p]
        cnt_v = cnt_vs[p]
        pltpu.sync_copy(tok_hbm.at[row], tok_v)
        pltpu.sync_copy(cnt_hbm.at[row], cnt_v)
        toks = {}
        cnts = {}
        dups = {}
        for d in range(_DEPTH):
            dbase = d * _NCHILD
            for c in range(4):
                t = tok_v[pl.ds(dbase + c * 16, 16)]
                toks[(d, c)] = t
                cnts[(d, c)] = cnt_v[pl.ds(dbase + c * 16, 16)]
                dup = lane < 0
                for j in range(1, 16):
                    bc = plsc.load_gather(
                        tok_v, [jnp.full((16,), dbase + c * 16 + j, jnp.int32)])
                    dup = jnp.logical_or(
                        dup, jnp.logical_and(t == bc, lane < j))
                dups[(d, c)] = dup
        for h in range(2):
            buf = bufs[h]
            lo = h * _HALF
            if handles[h] is not None:
                handles[h].wait()
                tv_old = tok_vs[(r - 1) % 2]
                for d in range(_DEPTH):
                    for c in range(4):
                        t_old = tv_old[pl.ds(d * _NCHILD + c * 16, 16)]
                        rel = jnp.minimum(jnp.maximum(t_old - lo, 0), _HALF - 1)
                        inr = jnp.logical_and(t_old >= lo, t_old < lo + _HALF)
                        plsc.store_scatter(buf, [rel], bv, mask=inr)
            for d in range(_DEPTH):
                rels = {}
                inrs = {}
                olds = {}
                for c in range(4):
                    t = toks[(d, c)]
                    rels[c] = jnp.minimum(jnp.maximum(t - lo, 0), _HALF - 1)
                    inrs[c] = jnp.logical_and(t >= lo, t < lo + _HALF)
                    olds[c] = plsc.load_gather(buf, [rels[c]])
                wd = w_v[d]
                for c in range(4):
                    new = olds[c] + wd * cnts[(d, c)]
                    keep = jnp.logical_and(inrs[c],
                                           jnp.logical_not(dups[(d, c)]))
                    plsc.store_scatter(buf, [rels[c]], new, mask=keep)
            handles[h] = pltpu.async_copy(
                buf, out_hbm.at[pl.ds(row * _VOCAB + lo, _HALF)], sems[h])
    handles[0].wait()
    handles[1].wait()


def _make_call():
    mesh = plsc.VectorSubcoreMesh(core_axis_name="c", subcore_axis_name="s")
    return pl.kernel(
        _sc_body,
        out_type=jax.ShapeDtypeStruct((_ROWS * _VOCAB,), jnp.float32),
        mesh=mesh,
        compiler_params=pltpu.CompilerParams(needs_layout_passes=False),
        scratch_types=[
            pltpu.VMEM((_HALF,), jnp.float32),
            pltpu.VMEM((_HALF,), jnp.float32),
            pltpu.VMEM((_UPD,), jnp.int32),
            pltpu.VMEM((_UPD,), jnp.int32),
            pltpu.VMEM((_UPD,), jnp.float32),
            pltpu.VMEM((_UPD,), jnp.float32),
            pltpu.VMEM((_DEPTH, 16), jnp.float32),
            pltpu.VMEM((16,), jnp.float32),
            pltpu.SemaphoreType.DMA,
            pltpu.SemaphoreType.DMA,
        ],
    )


def kernel(idx, child_tokens, counts, W, b_lin):
    del idx
    tok4 = child_tokens.reshape(_ROWS, _DEPTH, _NCHILD)
    cnt4 = counts.reshape(_ROWS, _DEPTH, _NCHILD).astype(jnp.float32)
    block = jnp.arange(_ROWS * _DEPTH, dtype=jnp.int32).reshape(_ROWS, _DEPTH, 1)
    keys = (block * _VOCAB + tok4).reshape(-1)
    keys_s, vals_s = lax.sort((keys, cnt4.reshape(-1)), dimension=0,
                              num_keys=1, is_stable=False)
    base_s = (jnp.arange(_ROWS * _UPD, dtype=jnp.int32) // _NCHILD) * _VOCAB
    tok = (keys_s - base_s).reshape(_ROWS, _UPD)
    cnt = vals_s.reshape(_ROWS, _UPD)
    wb = jnp.broadcast_to(W.reshape(_DEPTH, 1).astype(jnp.float32), (_DEPTH, 16))
    bb = jnp.broadcast_to(b_lin.reshape(1).astype(jnp.float32), (16,))
    out = _make_call()(tok, cnt, wb, bb)
    return out.reshape(_B, _T, _VOCAB)


# speedup vs baseline: 2.1196x; 2.1196x over previous
import jax
import jax.numpy as jnp
from jax import lax
from jax.experimental import pallas as pl
from jax.experimental.pallas import tpu as pltpu
from jax.experimental.pallas import tpu_sc as plsc

_VOCAB = 100000
_DEPTH = 4
_NCHILD = 64
_B, _T = 4, 32
_ROWS = _B * _T
_UPD = _DEPTH * _NCHILD
_NW = 32
_ROWS_PER_W = _ROWS // _NW
_FILL_UNROLL = 25
_FILL_STEPS = _VOCAB // (16 * _FILL_UNROLL)


def _sc_body(tok_hbm, cnt_hbm, w_hbm, b_hbm, out_hbm,
             row_v, tok_v, cnt_v, w_v, b_v):
    wid = lax.axis_index("s") * 2 + lax.axis_index("c")
    pltpu.sync_copy(w_hbm, w_v)
    pltpu.sync_copy(b_hbm, b_v)
    bv = b_v[...]
    lane = lax.iota(jnp.int32, 16)

    def _fill(i, c):
        base = i * (16 * _FILL_UNROLL)
        for u in range(_FILL_UNROLL):
            row_v[pl.ds(base + u * 16, 16)] = bv
        return c

    lax.fori_loop(0, _FILL_STEPS, _fill, 0)

    for r in range(_ROWS_PER_W):
        row = wid * _ROWS_PER_W + r
        pltpu.sync_copy(tok_hbm.at[row], tok_v)
        pltpu.sync_copy(cnt_hbm.at[row], cnt_v)
        for d in range(_DEPTH):
            dbase = d * _NCHILD
            toks = [tok_v[pl.ds(dbase + c * 16, 16)] for c in range(4)]
            cnts = [cnt_v[pl.ds(dbase + c * 16, 16)] for c in range(4)]
            if d == 0:
                olds = [bv] * 4
            else:
                olds = [plsc.load_gather(row_v, [toks[c]]) for c in range(4)]
            wd = w_v[d]
            news = [olds[c] + wd * cnts[c] for c in range(4)]
            for c in range(4):
                dup = lane < 0
                for j in range(1, 16):
                    bc = plsc.load_gather(
                        tok_v, [jnp.full((16,), dbase + c * 16 + j, jnp.int32)])
                    dup = jnp.logical_or(
                        dup, jnp.logical_and(toks[c] == bc, lane < j))
                plsc.store_scatter(row_v, [toks[c]], news[c],
                                   mask=jnp.logical_not(dup))
        pltpu.sync_copy(row_v, out_hbm.at[row])
        for d in range(_DEPTH):
            for c in range(4):
                tc = tok_v[pl.ds(d * _NCHILD + c * 16, 16)]
                plsc.store_scatter(row_v, [tc], bv)


def _make_call():
    mesh = plsc.VectorSubcoreMesh(core_axis_name="c", subcore_axis_name="s")
    return pl.kernel(
        _sc_body,
        out_type=jax.ShapeDtypeStruct((_ROWS, _VOCAB), jnp.float32),
        mesh=mesh,
        compiler_params=pltpu.CompilerParams(needs_layout_passes=False),
        scratch_types=[
            pltpu.VMEM((_VOCAB,), jnp.float32),
            pltpu.VMEM((_UPD,), jnp.int32),
            pltpu.VMEM((_UPD,), jnp.float32),
            pltpu.VMEM((_DEPTH, 16), jnp.float32),
            pltpu.VMEM((16,), jnp.float32),
        ],
    )


def kernel(idx, child_tokens, counts, W, b_lin):
    del idx
    tok4 = child_tokens.reshape(_ROWS, _DEPTH, _NCHILD)
    cnt4 = counts.reshape(_ROWS, _DEPTH, _NCHILD).astype(jnp.float32)
    block = jnp.arange(_ROWS * _DEPTH, dtype=jnp.int32).reshape(_ROWS, _DEPTH, 1)
    keys = (block * _VOCAB + tok4).reshape(-1)
    keys_s, vals_s = lax.sort((keys, cnt4.reshape(-1)), dimension=0,
                              num_keys=1, is_stable=False)
    base_s = (jnp.arange(_ROWS * _UPD, dtype=jnp.int32) // _NCHILD) * _VOCAB
    tok = (keys_s - base_s).reshape(_ROWS, _UPD)
    cnt = vals_s.reshape(_ROWS, _UPD)
    wb = jnp.broadcast_to(W.reshape(_DEPTH, 1).astype(jnp.float32), (_DEPTH, 16))
    bb = jnp.broadcast_to(b_lin.reshape(1).astype(jnp.float32), (16,))
    out = _make_call()(tok, cnt, wb, bb)
    return out.reshape(_B, _T, _VOCAB)


# speedup vs baseline: 2.1545x; 1.0165x over previous
import jax
import jax.numpy as jnp
from jax import lax
from jax.experimental import pallas as pl
from jax.experimental.pallas import tpu as pltpu
from jax.experimental.pallas import tpu_sc as plsc

_VOCAB = 100000
_DEPTH = 4
_NCHILD = 64
_B, _T = 4, 32
_ROWS = _B * _T
_UPD = _DEPTH * _NCHILD
_NW = 32
_ROWS_PER_W = _ROWS // _NW
_FILL_UNROLL = 25
_FILL_STEPS = _VOCAB // (16 * _FILL_UNROLL)


def _sc_body(key_hbm, val_hbm, b_hbm, out_hbm,
             row_v, key_v, val_v, b_v):
    wid = lax.axis_index("s") * 2 + lax.axis_index("c")
    pltpu.sync_copy(b_hbm, b_v)
    bv = b_v[...]
    lane = lax.iota(jnp.int32, 16)

    def _fill(i, c):
        base = i * (16 * _FILL_UNROLL)
        for u in range(_FILL_UNROLL):
            row_v[pl.ds(base + u * 16, 16)] = bv
        return c

    lax.fori_loop(0, _FILL_STEPS, _fill, 0)

    for r in range(_ROWS_PER_W):
        row = wid * _ROWS_PER_W + r
        pltpu.sync_copy(key_hbm.at[row], key_v)
        pltpu.sync_copy(val_hbm.at[row], val_v)
        for d in range(_DEPTH):
            dbase = d * _NCHILD
            kbase = (row * _DEPTH + d) * _VOCAB
            raws = [key_v[pl.ds(dbase + c * 16, 16)] for c in range(4)]
            toks = [raws[c] - kbase for c in range(4)]
            cnts = [val_v[pl.ds(dbase + c * 16, 16)] for c in range(4)]
            if d == 0:
                olds = [bv] * 4
            else:
                olds = [plsc.load_gather(row_v, [toks[c]]) for c in range(4)]
            news = [olds[c] + cnts[c] for c in range(4)]
            for c in range(4):
                dup = lane < 0
                for j in range(1, 16):
                    bc = plsc.load_gather(
                        key_v, [jnp.full((16,), dbase + c * 16 + j, jnp.int32)])
                    dup = jnp.logical_or(
                        dup, jnp.logical_and(raws[c] == bc, lane < j))
                plsc.store_scatter(row_v, [toks[c]], news[c],
                                   mask=jnp.logical_not(dup))
        pltpu.sync_copy(row_v, out_hbm.at[row])
        for d in range(_DEPTH):
            kbase = (row * _DEPTH + d) * _VOCAB
            for c in range(4):
                tc = key_v[pl.ds(d * _NCHILD + c * 16, 16)] - kbase
                plsc.store_scatter(row_v, [tc], bv)


def _make_call():
    mesh = plsc.VectorSubcoreMesh(core_axis_name="c", subcore_axis_name="s")
    return pl.kernel(
        _sc_body,
        out_type=jax.ShapeDtypeStruct((_ROWS, _VOCAB), jnp.float32),
        mesh=mesh,
        compiler_params=pltpu.CompilerParams(needs_layout_passes=False),
        scratch_types=[
            pltpu.VMEM((_VOCAB,), jnp.float32),
            pltpu.VMEM((_UPD,), jnp.int32),
            pltpu.VMEM((_UPD,), jnp.float32),
            pltpu.VMEM((16,), jnp.float32),
        ],
    )


def kernel(idx, child_tokens, counts, W, b_lin):
    del idx
    tok4 = child_tokens.reshape(_ROWS, _DEPTH, _NCHILD)
    cnt4 = counts.reshape(_ROWS, _DEPTH, _NCHILD).astype(jnp.float32)
    block = jnp.arange(_ROWS * _DEPTH, dtype=jnp.int32).reshape(_ROWS, _DEPTH, 1)
    keys = (block * _VOCAB + tok4).reshape(-1)
    vals = (cnt4 * W.reshape(1, _DEPTH, 1).astype(jnp.float32)).reshape(-1)
    keys_s, vals_s = lax.sort((keys, vals), dimension=0,
                              num_keys=1, is_stable=False)
    key2 = keys_s.reshape(_ROWS, _UPD)
    val2 = vals_s.reshape(_ROWS, _UPD)
    bb = jnp.broadcast_to(b_lin.reshape(1).astype(jnp.float32), (16,))
    out = _make_call()(key2, val2, bb)
    return out.reshape(_B, _T, _VOCAB)
